# baseline (speedup 1.0000x reference)
import functools

import jax
import jax.numpy as jnp
from jax import lax
from jax.experimental import pallas as pl
from jax.experimental.pallas import tpu as pltpu
from jax.experimental.pallas import tpu_sc as plsc

N = 10000
E = 160000
NC = 2
NS = 16
NW = NC * NS
CHUNK = 128
CHUNKS_PER_TILE = 40
E_PAD = NW * CHUNKS_PER_TILE * CHUNK
N_PAD = 10112
TILE_ROWS = N_PAD // NS

D1, D2, D3, D4 = 170, 113, 56, 1
W1A, W1B = 96, 80
W2, W3, W4 = 128, 64, 16

_GRID = 10
_BN = N // _GRID


def _make_sc_segment_sum(width, with_count=False):
    mesh = plsc.VectorSubcoreMesh(core_axis_name="c", subcore_axis_name="s")
    out_type = [jax.ShapeDtypeStruct((NC, N_PAD, width), jnp.float32)]
    nbuf = 4 if width <= 80 else 2
    scratch = [
        pltpu.VMEM((CHUNKS_PER_TILE, CHUNK), jnp.int32),
        pltpu.VMEM((CHUNKS_PER_TILE, CHUNK), jnp.int32),
    ] + [pltpu.VMEM((CHUNK, width), jnp.float32) for _ in range(nbuf)] + [
        pltpu.VMEM_SHARED((N_PAD, width), jnp.float32),
    ] + [pltpu.SemaphoreType.DMA for _ in range(2 * nbuf)]
    if with_count:
        out_type.append(jax.ShapeDtypeStruct((NW, N_PAD), jnp.float32))
        scratch.append(pltpu.VMEM((N_PAD,), jnp.float32))

    def body(*refs):
        if with_count:
            (y_hbm, src_hbm, dst_hbm, zacc_hbm, zrow_hbm,
             out_hbm, cnt_hbm, srcs_v, dsts_v) = refs[:9]
            rows = refs[9:9 + nbuf]
            acc_sh = refs[9 + nbuf]
            gsem = refs[10 + nbuf:10 + 2 * nbuf]
            ssem = refs[10 + 2 * nbuf:10 + 3 * nbuf]
            cnt_v = refs[10 + 3 * nbuf]
        else:
            (y_hbm, src_hbm, dst_hbm, zacc_hbm, out_hbm,
             srcs_v, dsts_v) = refs[:7]
            rows = refs[7:7 + nbuf]
            acc_sh = refs[7 + nbuf]
            gsem = refs[8 + nbuf:8 + 2 * nbuf]
            ssem = refs[8 + 2 * nbuf:8 + 3 * nbuf]

        cid = lax.axis_index("c")
        sid = lax.axis_index("s")
        wid = sid * NC + cid

        r0 = pl.multiple_of(sid * TILE_ROWS, 8)
        pltpu.sync_copy(zacc_hbm.at[pl.ds(r0, TILE_ROWS)],
                        acc_sh.at[pl.ds(r0, TILE_ROWS)])
        cb = pl.multiple_of(wid * CHUNKS_PER_TILE, 8)
        pltpu.sync_copy(src_hbm.at[pl.ds(cb, CHUNKS_PER_TILE)], srcs_v)
        pltpu.sync_copy(dst_hbm.at[pl.ds(cb, CHUNKS_PER_TILE)], dsts_v)
        if with_count:
            pltpu.sync_copy(zrow_hbm, cnt_v)
        plsc.subcore_barrier()

        if with_count:
            ones16 = jnp.full((16,), 1.0, jnp.float32)
            for cc in range(CHUNKS_PER_TILE):
                for kk in range(CHUNK // 16):
                    d_vec = dsts_v[cc, pl.ds(kk * 16, 16)]
                    plsc.addupdate_scatter(cnt_v, [d_vec], ones16)

        def _gather(c, b):
            pltpu.async_copy(y_hbm.at[srcs_v.at[c]], rows[b], gsem[b])

        def _gather_wait(c, b):
            pltpu.make_async_copy(y_hbm.at[srcs_v.at[c]], rows[b],
                                  gsem[b]).wait()

        def _scatter(c, b):
            pltpu.async_copy(rows[b], acc_sh.at[dsts_v.at[c]], ssem[b],
                             add=True)

        def _scatter_wait(c, b):
            pltpu.make_async_copy(rows[b], acc_sh.at[dsts_v.at[c]],
                                  ssem[b]).wait()

        if nbuf == 2:
            _gather(0, 0)
            _gather(1, 1)

            def step(i, carry):
                for b in range(2):
                    c = 2 * i + b
                    _gather_wait(c, b)
                    _scatter(c, b)
                    _scatter_wait(c, b)
                    _gather(c + 2, b)
                return carry

            lax.fori_loop(0, CHUNKS_PER_TILE // 2 - 1, step, 0)
            for b in range(2):
                c = CHUNKS_PER_TILE - 2 + b
                _gather_wait(c, b)
                _scatter(c, b)
                _scatter_wait(c, b)
        else:
            for b in range(nbuf - 1):
                _gather(b, b)

            def chunk_body(c, b, do_sw, do_gi):
                _gather_wait(c, b)
                _scatter(c, b)
                if do_sw:
                    _scatter_wait(c - 1, (b - 1) % nbuf)
                if do_gi:
                    _gather(c + nbuf - 1, (b + nbuf - 1) % nbuf)

            for c in range(nbuf):
                chunk_body(c, c % nbuf, c >= 1, True)

            def step(i, carry):
                for b in range(nbuf):
                    chunk_body(nbuf * i + b, b, True, True)
                return carry

            lax.fori_loop(1, CHUNKS_PER_TILE // nbuf - 1, step, 0)

            for c in range(CHUNKS_PER_TILE - nbuf, CHUNKS_PER_TILE):
                chunk_body(c, c % nbuf, True, c <= CHUNKS_PER_TILE - nbuf)
            _scatter_wait(CHUNKS_PER_TILE - 1, (CHUNKS_PER_TILE - 1) % nbuf)

        plsc.subcore_barrier()

        pltpu.sync_copy(acc_sh.at[pl.ds(r0, TILE_ROWS)],
                        out_hbm.at[cid].at[pl.ds(r0, TILE_ROWS)])
        if with_count:
            pltpu.sync_copy(cnt_v, cnt_hbm.at[wid])

    outs = tuple(out_type) if with_count else out_type[0]
    return pl.kernel(body, out_type=outs, mesh=mesh, scratch_types=scratch,
                     compiler_params=pltpu.CompilerParams(
                         use_tc_tiling_on_sc=False,
                         needs_layout_passes=False))


def _make_sc_scalar_segment_sum():
    mesh = plsc.VectorSubcoreMesh(core_axis_name="c", subcore_axis_name="s")
    out_type = jax.ShapeDtypeStruct((NW, N_PAD), jnp.float32)
    scratch = [
        pltpu.VMEM((CHUNKS_PER_TILE, CHUNK), jnp.int32),
        pltpu.VMEM((CHUNKS_PER_TILE, CHUNK), jnp.int32),
        pltpu.VMEM((N,), jnp.float32),
        pltpu.VMEM((N_PAD,), jnp.float32),
    ]

    def body(y_hbm, src_hbm, dst_hbm, zrow_hbm, out_hbm,
             srcs_v, dsts_v, y_v, acc_v):
        cid = lax.axis_index("c")
        sid = lax.axis_index("s")
        wid = sid * NC + cid
        cb = pl.multiple_of(wid * CHUNKS_PER_TILE, 8)
        pltpu.sync_copy(src_hbm.at[pl.ds(cb, CHUNKS_PER_TILE)], srcs_v)
        pltpu.sync_copy(dst_hbm.at[pl.ds(cb, CHUNKS_PER_TILE)], dsts_v)
        pltpu.sync_copy(y_hbm, y_v)
        pltpu.sync_copy(zrow_hbm, acc_v)

        for cc in range(CHUNKS_PER_TILE):
            for kk in range(CHUNK // 16):
                s_vec = srcs_v[cc, pl.ds(kk * 16, 16)]
                d_vec = dsts_v[cc, pl.ds(kk * 16, 16)]
                vals = plsc.load_gather(y_v, [s_vec])
                plsc.addupdate_scatter(acc_v, [d_vec], vals)

        pltpu.sync_copy(acc_v, out_hbm.at[wid])

    return pl.kernel(body, out_type=out_type, mesh=mesh,
                     scratch_types=scratch,
                     compiler_params=pltpu.CompilerParams(
                         use_tc_tiling_on_sc=False,
                         needs_layout_passes=False))


COLS = 4


def _make_sc_colsplit_segment_sum():
    mesh = plsc.VectorSubcoreMesh(core_axis_name="c", subcore_axis_name="s")
    out_type = jax.ShapeDtypeStruct((NW, COLS * N_PAD), jnp.float32)
    nchunks = E_PAD // CHUNK
    GW = CHUNK * COLS
    scratch = [
        pltpu.VMEM((COLS * N,), jnp.float32),
        pltpu.VMEM((COLS * N_PAD,), jnp.float32),
        pltpu.VMEM((GW,), jnp.int32),
        pltpu.VMEM((GW,), jnp.int32),
        pltpu.VMEM((GW,), jnp.int32),
        pltpu.VMEM((GW,), jnp.int32),
        pltpu.SemaphoreType.DMA,
        pltpu.SemaphoreType.DMA,
    ]

    def body(yblk_hbm, sx_hbm, dx_hbm, zcol_hbm, out_hbm,
             y_v, acc_v, si0, di0, si1, di1, gs0, gs1):
        cid = lax.axis_index("c")
        sid = lax.axis_index("s")
        wid = sid * NC + cid
        sibuf = (si0, si1)
        dibuf = (di0, di1)
        sems = (gs0, gs1)
        pltpu.sync_copy(yblk_hbm.at[wid], y_v)
        pltpu.sync_copy(zcol_hbm, acc_v)

        def idx_dma(c, b):
            off = pl.multiple_of(c * GW, 8)
            pltpu.async_copy(sx_hbm.at[pl.ds(off, GW)], sibuf[b], sems[b])
            pltpu.async_copy(dx_hbm.at[pl.ds(off, GW)], dibuf[b], sems[b])

        def idx_wait(b):
            pltpu.make_async_copy(sx_hbm.at[pl.ds(0, GW)], sibuf[b],
                                  sems[b]).wait()
            pltpu.make_async_copy(dx_hbm.at[pl.ds(0, GW)], dibuf[b],
                                  sems[b]).wait()

        def compute(b):
            for g in range(GW // 16):
                sv = sibuf[b][pl.ds(g * 16, 16)]
                dv = dibuf[b][pl.ds(g * 16, 16)]
                vals = plsc.load_gather(y_v, [sv])
                plsc.addupdate_scatter(acc_v, [dv], vals)

        idx_dma(0, 0)

        def step(i, carry):
            for b in range(2):
                c = 2 * i + b
                idx_wait(b)
                idx_dma(c + 1, 1 - b)
                compute(b)
            return carry

        lax.fori_loop(0, nchunks // 2 - 1, step, 0)
        idx_wait(0)
        idx_dma(nchunks - 1, 1)
        compute(0)
        idx_wait(1)
        compute(1)

        pltpu.sync_copy(acc_v, out_hbm.at[wid])

    return pl.kernel(body, out_type=out_type, mesh=mesh,
                     scratch_types=scratch,
                     compiler_params=pltpu.CompilerParams(
                         use_tc_tiling_on_sc=False,
                         needs_layout_passes=False))


_sc_layer1a = _make_sc_segment_sum(W1A, with_count=True)
_sc_layer1b = _make_sc_segment_sum(W1B)
_sc_layer2 = _make_sc_colsplit_segment_sum()
_sc_layer3 = _make_sc_segment_sum(W3)
_sc_layer4 = _make_sc_scalar_segment_sum()


def _tc_first_body(x_ref, wla_ref, wlb_ref, wr_ref, bl_ref,
                   ya_ref, yb_ref, z_ref):
    x = x_ref[...]
    ya_ref[...] = jnp.dot(x, wla_ref[...], preferred_element_type=jnp.float32)
    yb_ref[...] = jnp.dot(x, wlb_ref[...], preferred_element_type=jnp.float32)
    z_ref[...] = (jnp.dot(x, wr_ref[...], preferred_element_type=jnp.float32)
                  + bl_ref[...])


def _tc_first(x, wla, wlb, wr, bl):
    d_in = x.shape[1]
    d_out = wr.shape[1]
    return pl.pallas_call(
        _tc_first_body,
        grid=(_GRID,),
        in_specs=[
            pl.BlockSpec((_BN, d_in), lambda i: (i, 0)),
            pl.BlockSpec((d_in, W1A), lambda i: (0, 0)),
            pl.BlockSpec((d_in, W1B), lambda i: (0, 0)),
            pl.BlockSpec((d_in, d_out), lambda i: (0, 0)),
            pl.BlockSpec((1, d_out), lambda i: (0, 0)),
        ],
        out_specs=[
            pl.BlockSpec((_BN, W1A), lambda i: (i, 0)),
            pl.BlockSpec((_BN, W1B), lambda i: (i, 0)),
            pl.BlockSpec((_BN, d_out), lambda i: (i, 0)),
        ],
        out_shape=[
            jax.ShapeDtypeStruct((N, W1A), jnp.float32),
            jax.ShapeDtypeStruct((N, W1B), jnp.float32),
            jax.ShapeDtypeStruct((N, d_out), jnp.float32),
        ],
    )(x, wla, wlb, wr, bl)


def _combine(p_refs, takes, cnt_ref, z_ref):
    cnt = jnp.sum(cnt_ref[...], axis=1, keepdims=True)
    inv = 1.0 / jnp.maximum(cnt, 1.0)
    parts = [(p[0, :, :t] + p[1, :, :t]) if p.ndim == 3 else p[:, :t]
             for p, t in zip(p_refs, takes)]
    agg = parts[0] if len(parts) == 1 else jnp.concatenate(parts, axis=1)
    return agg * inv + z_ref[...]


def _tc_mid_body(takes, *refs):
    np_ = len(takes)
    p_refs = refs[:np_]
    cnt_ref, z_ref, wl_ref, wr_ref, bl_ref, y_ref, z_out_ref = refs[np_:]
    x = _combine(p_refs, takes, cnt_ref, z_ref)
    y_ref[...] = jnp.dot(x, wl_ref[...], preferred_element_type=jnp.float32)
    z_out_ref[...] = (jnp.dot(x, wr_ref[...],
                              preferred_element_type=jnp.float32) + bl_ref[...])


def _tc_mid(p_parts, takes, cnt, z, wl_pad, wr, bl):
    d_prev = z.shape[1]
    wp, d_out = wl_pad.shape[1], wr.shape[1]
    p_specs = [pl.BlockSpec((NC, _BN, p.shape[2]), lambda i: (0, i, 0))
               if p.ndim == 3 else
               pl.BlockSpec((_BN, p.shape[1]), lambda i: (i, 0))
               for p in p_parts]
    return pl.pallas_call(
        functools.partial(_tc_mid_body, tuple(takes)),
        grid=(_GRID,),
        in_specs=p_specs + [
            pl.BlockSpec((_BN, NW), lambda i: (i, 0)),
            pl.BlockSpec((_BN, d_prev), lambda i: (i, 0)),
            pl.BlockSpec((d_prev, wp), lambda i: (0, 0)),
            pl.BlockSpec((d_prev, d_out), lambda i: (0, 0)),
            pl.BlockSpec((1, d_out), lambda i: (0, 0)),
        ],
        out_specs=[
            pl.BlockSpec((_BN, wp), lambda i: (i, 0)),
            pl.BlockSpec((_BN, d_out), lambda i: (i, 0)),
        ],
        out_shape=[
            jax.ShapeDtypeStruct((N, wp), jnp.float32),
            jax.ShapeDtypeStruct((N, d_out), jnp.float32),
        ],
    )(*p_parts, cnt, z, wl_pad, wr, bl)


def _tc_last_body(p_ref, cnt_ref, z_ref, out_ref):
    cnt = jnp.sum(cnt_ref[...], axis=1, keepdims=True)
    inv = 1.0 / jnp.maximum(cnt, 1.0)
    agg = jnp.sum(p_ref[...], axis=1, keepdims=True)
    out_ref[...] = agg * inv + z_ref[...]


def _tc_last(p, cnt, z):
    return pl.pallas_call(
        _tc_last_body,
        grid=(_GRID,),
        in_specs=[
            pl.BlockSpec((_BN, NW), lambda i: (i, 0)),
            pl.BlockSpec((_BN, NW), lambda i: (i, 0)),
            pl.BlockSpec((_BN, D4), lambda i: (i, 0)),
        ],
        out_specs=pl.BlockSpec((_BN, D4), lambda i: (i, 0)),
        out_shape=jax.ShapeDtypeStruct((N, D4), jnp.float32),
    )(p, cnt, z)


def _pad_w(w, width):
    return jnp.pad(w, ((0, 0), (0, width - w.shape[1])))


def kernel(x, edge_index, Wl1, bl1, Wr1, Wl2, bl2, Wr2, Wl3, bl3, Wr3,
           Wl4, bl4, Wr4):
    src = edge_index[0].astype(jnp.int32)
    dst = edge_index[1].astype(jnp.int32)
    pad = E_PAD - E
    src = jnp.concatenate([src, jnp.zeros((pad,), jnp.int32)])
    dst = jnp.concatenate([dst, jnp.full((pad,), N, jnp.int32)])
    sx = (src[:, None] * COLS + jnp.arange(COLS, dtype=jnp.int32)).reshape(-1)
    dx = (dst[:, None] * COLS + jnp.arange(COLS, dtype=jnp.int32)).reshape(-1)
    src = src.reshape(E_PAD // CHUNK, CHUNK)
    dst = dst.reshape(E_PAD // CHUNK, CHUNK)

    zeros = jnp.zeros((N_PAD, W2), jnp.float32)
    zrow = jnp.zeros((N_PAD,), jnp.float32)
    zcol = jnp.zeros((COLS * N_PAD,), jnp.float32)

    wla = Wl1[:, :W1A]
    wlb = _pad_w(Wl1[:, W1A:], W1B)

    y1a, y1b, z1 = _tc_first(x, wla, wlb, Wr1, bl1.reshape(1, -1))
    p1a, cnt = _sc_layer1a(y1a, src, dst, zeros[:, :W1A], zrow)
    p1b = _sc_layer1b(y1b, src, dst, zeros[:, :W1B])
    cnt = cnt.T[:N]

    y2, z2 = _tc_mid([p1a[:, :N], p1b[:, :N]], (W1A, D1 - W1A), cnt, z1,
                     _pad_w(Wl2, W2), Wr2, bl2.reshape(1, -1))
    y2blk = y2.reshape(N, NW, COLS).transpose(1, 0, 2).reshape(NW, COLS * N)
    p2 = _sc_layer2(y2blk, sx, dx, zcol)
    p2r = (p2.reshape(NW, N_PAD, COLS)[:, :N]
           .transpose(1, 0, 2).reshape(N, W2))

    y3, z3 = _tc_mid([p2r], (D2,), cnt, z2,
                     _pad_w(Wl3, W3), Wr3, bl3.reshape(1, -1))
    p3 = _sc_layer3(y3, src, dst, zeros[:, :W3])

    y4, z4 = _tc_mid([p3[:, :N]], (D3,), cnt, z3,
                     Wl4, Wr4, bl4.reshape(1, -1))
    p4 = _sc_layer4(y4.reshape(N), src, dst, zrow)

    return _tc_last(p4.T[:N], cnt, z4)


# speedup vs baseline: 1.7843x; 1.7843x over previous
import functools

import jax
import jax.numpy as jnp
from jax import lax
from jax.experimental import pallas as pl
from jax.experimental.pallas import tpu as pltpu
from jax.experimental.pallas import tpu_sc as plsc

N = 10000
E = 160000
NC = 2
NS = 16
NW = NC * NS
CHUNK = 128
CHUNKS_PER_TILE = 40
E_PAD = NW * CHUNKS_PER_TILE * CHUNK
N_PAD = 10112
TILE_ROWS = N_PAD // NS

D1, D2, D3, D4 = 170, 113, 56, 1
W1A, W1B = 96, 80
W2, W3, W4 = 128, 64, 16

_GRID = 10
_BN = N // _GRID


def _make_sc_segment_sum(width, with_count=False):
    mesh = plsc.VectorSubcoreMesh(core_axis_name="c", subcore_axis_name="s")
    out_type = [jax.ShapeDtypeStruct((NC, N_PAD, width), jnp.float32)]
    nbuf = 4 if width <= 80 else 2
    scratch = [
        pltpu.VMEM((CHUNKS_PER_TILE, CHUNK), jnp.int32),
        pltpu.VMEM((CHUNKS_PER_TILE, CHUNK), jnp.int32),
    ] + [pltpu.VMEM((CHUNK, width), jnp.float32) for _ in range(nbuf)] + [
        pltpu.VMEM_SHARED((N_PAD, width), jnp.float32),
    ] + [pltpu.SemaphoreType.DMA for _ in range(2 * nbuf)]
    if with_count:
        out_type.append(jax.ShapeDtypeStruct((NW, N_PAD), jnp.float32))
        scratch.append(pltpu.VMEM((N_PAD,), jnp.float32))

    def body(*refs):
        if with_count:
            (y_hbm, src_hbm, dst_hbm, zacc_hbm, zrow_hbm,
             out_hbm, cnt_hbm, srcs_v, dsts_v) = refs[:9]
            rows = refs[9:9 + nbuf]
            acc_sh = refs[9 + nbuf]
            gsem = refs[10 + nbuf:10 + 2 * nbuf]
            ssem = refs[10 + 2 * nbuf:10 + 3 * nbuf]
            cnt_v = refs[10 + 3 * nbuf]
        else:
            (y_hbm, src_hbm, dst_hbm, zacc_hbm, out_hbm,
             srcs_v, dsts_v) = refs[:7]
            rows = refs[7:7 + nbuf]
            acc_sh = refs[7 + nbuf]
            gsem = refs[8 + nbuf:8 + 2 * nbuf]
            ssem = refs[8 + 2 * nbuf:8 + 3 * nbuf]

        cid = lax.axis_index("c")
        sid = lax.axis_index("s")
        wid = sid * NC + cid

        r0 = pl.multiple_of(sid * TILE_ROWS, 8)
        pltpu.sync_copy(zacc_hbm.at[pl.ds(r0, TILE_ROWS)],
                        acc_sh.at[pl.ds(r0, TILE_ROWS)])
        cb = pl.multiple_of(wid * CHUNKS_PER_TILE, 8)
        pltpu.sync_copy(src_hbm.at[pl.ds(cb, CHUNKS_PER_TILE)], srcs_v)
        pltpu.sync_copy(dst_hbm.at[pl.ds(cb, CHUNKS_PER_TILE)], dsts_v)
        if with_count:
            pltpu.sync_copy(zrow_hbm, cnt_v)
        plsc.subcore_barrier()

        if with_count:
            ones16 = jnp.full((16,), 1.0, jnp.float32)
            for cc in range(CHUNKS_PER_TILE):
                for kk in range(CHUNK // 16):
                    d_vec = dsts_v[cc, pl.ds(kk * 16, 16)]
                    plsc.addupdate_scatter(cnt_v, [d_vec], ones16)

        def _gather(c, b):
            pltpu.async_copy(y_hbm.at[srcs_v.at[c]], rows[b], gsem[b])

        def _gather_wait(c, b):
            pltpu.make_async_copy(y_hbm.at[srcs_v.at[c]], rows[b],
                                  gsem[b]).wait()

        def _scatter(c, b):
            pltpu.async_copy(rows[b], acc_sh.at[dsts_v.at[c]], ssem[b],
                             add=True)

        def _scatter_wait(c, b):
            pltpu.make_async_copy(rows[b], acc_sh.at[dsts_v.at[c]],
                                  ssem[b]).wait()

        if nbuf == 2:
            _gather(0, 0)
            _gather(1, 1)

            def step(i, carry):
                for b in range(2):
                    c = 2 * i + b
                    _gather_wait(c, b)
                    _scatter(c, b)
                    _scatter_wait(c, b)
                    _gather(c + 2, b)
                return carry

            lax.fori_loop(0, CHUNKS_PER_TILE // 2 - 1, step, 0)
            for b in range(2):
                c = CHUNKS_PER_TILE - 2 + b
                _gather_wait(c, b)
                _scatter(c, b)
                _scatter_wait(c, b)
        else:
            for b in range(nbuf - 1):
                _gather(b, b)

            def chunk_body(c, b, do_sw, do_gi):
                _gather_wait(c, b)
                _scatter(c, b)
                if do_sw:
                    _scatter_wait(c - 1, (b - 1) % nbuf)
                if do_gi:
                    _gather(c + nbuf - 1, (b + nbuf - 1) % nbuf)

            for c in range(nbuf):
                chunk_body(c, c % nbuf, c >= 1, True)

            def step(i, carry):
                for b in range(nbuf):
                    chunk_body(nbuf * i + b, b, True, True)
                return carry

            lax.fori_loop(1, CHUNKS_PER_TILE // nbuf - 1, step, 0)

            for c in range(CHUNKS_PER_TILE - nbuf, CHUNKS_PER_TILE):
                chunk_body(c, c % nbuf, True, c <= CHUNKS_PER_TILE - nbuf)
            _scatter_wait(CHUNKS_PER_TILE - 1, (CHUNKS_PER_TILE - 1) % nbuf)

        plsc.subcore_barrier()

        pltpu.sync_copy(acc_sh.at[pl.ds(r0, TILE_ROWS)],
                        out_hbm.at[cid].at[pl.ds(r0, TILE_ROWS)])
        if with_count:
            pltpu.sync_copy(cnt_v, cnt_hbm.at[wid])

    outs = tuple(out_type) if with_count else out_type[0]
    return pl.kernel(body, out_type=outs, mesh=mesh, scratch_types=scratch,
                     compiler_params=pltpu.CompilerParams(
                         use_tc_tiling_on_sc=False,
                         needs_layout_passes=False))


def _make_sc_scalar_segment_sum():
    mesh = plsc.VectorSubcoreMesh(core_axis_name="c", subcore_axis_name="s")
    out_type = jax.ShapeDtypeStruct((NW, N_PAD), jnp.float32)
    scratch = [
        pltpu.VMEM((CHUNKS_PER_TILE, CHUNK), jnp.int32),
        pltpu.VMEM((CHUNKS_PER_TILE, CHUNK), jnp.int32),
        pltpu.VMEM((N,), jnp.float32),
        pltpu.VMEM((N_PAD,), jnp.float32),
    ]

    def body(y_hbm, src_hbm, dst_hbm, zrow_hbm, out_hbm,
             srcs_v, dsts_v, y_v, acc_v):
        cid = lax.axis_index("c")
        sid = lax.axis_index("s")
        wid = sid * NC + cid
        cb = pl.multiple_of(wid * CHUNKS_PER_TILE, 8)
        pltpu.sync_copy(src_hbm.at[pl.ds(cb, CHUNKS_PER_TILE)], srcs_v)
        pltpu.sync_copy(dst_hbm.at[pl.ds(cb, CHUNKS_PER_TILE)], dsts_v)
        pltpu.sync_copy(y_hbm, y_v)
        pltpu.sync_copy(zrow_hbm, acc_v)

        for cc in range(CHUNKS_PER_TILE):
            for kk in range(CHUNK // 16):
                s_vec = srcs_v[cc, pl.ds(kk * 16, 16)]
                d_vec = dsts_v[cc, pl.ds(kk * 16, 16)]
                vals = plsc.load_gather(y_v, [s_vec])
                plsc.addupdate_scatter(acc_v, [d_vec], vals)

        pltpu.sync_copy(acc_v, out_hbm.at[wid])

    return pl.kernel(body, out_type=out_type, mesh=mesh,
                     scratch_types=scratch,
                     compiler_params=pltpu.CompilerParams(
                         use_tc_tiling_on_sc=False,
                         needs_layout_passes=False))


_sc_layer1a = _make_sc_segment_sum(W1A, with_count=True)
_sc_layer1b = _make_sc_segment_sum(W1B)
_sc_layer2 = _make_sc_segment_sum(W2)
_sc_layer3 = _make_sc_segment_sum(W3)
_sc_layer4 = _make_sc_scalar_segment_sum()


def _tc_first_body(x_ref, wla_ref, wlb_ref, wr_ref, bl_ref,
                   ya_ref, yb_ref, z_ref):
    x = x_ref[...]
    ya_ref[...] = jnp.dot(x, wla_ref[...], preferred_element_type=jnp.float32)
    yb_ref[...] = jnp.dot(x, wlb_ref[...], preferred_element_type=jnp.float32)
    z_ref[...] = (jnp.dot(x, wr_ref[...], preferred_element_type=jnp.float32)
                  + bl_ref[...])


def _tc_first(x, wla, wlb, wr, bl):
    d_in = x.shape[1]
    d_out = wr.shape[1]
    return pl.pallas_call(
        _tc_first_body,
        grid=(_GRID,),
        in_specs=[
            pl.BlockSpec((_BN, d_in), lambda i: (i, 0)),
            pl.BlockSpec((d_in, W1A), lambda i: (0, 0)),
            pl.BlockSpec((d_in, W1B), lambda i: (0, 0)),
            pl.BlockSpec((d_in, d_out), lambda i: (0, 0)),
            pl.BlockSpec((1, d_out), lambda i: (0, 0)),
        ],
        out_specs=[
            pl.BlockSpec((_BN, W1A), lambda i: (i, 0)),
            pl.BlockSpec((_BN, W1B), lambda i: (i, 0)),
            pl.BlockSpec((_BN, d_out), lambda i: (i, 0)),
        ],
        out_shape=[
            jax.ShapeDtypeStruct((N, W1A), jnp.float32),
            jax.ShapeDtypeStruct((N, W1B), jnp.float32),
            jax.ShapeDtypeStruct((N, d_out), jnp.float32),
        ],
    )(x, wla, wlb, wr, bl)


def _combine(p_refs, takes, cnt_ref, z_ref):
    cnt = jnp.sum(cnt_ref[...], axis=1, keepdims=True)
    inv = 1.0 / jnp.maximum(cnt, 1.0)
    parts = [p[0, :, :t] + p[1, :, :t] for p, t in zip(p_refs, takes)]
    agg = parts[0] if len(parts) == 1 else jnp.concatenate(parts, axis=1)
    return agg * inv + z_ref[...]


def _tc_mid_body(takes, *refs):
    np_ = len(takes)
    p_refs = refs[:np_]
    cnt_ref, z_ref, wl_ref, wr_ref, bl_ref, y_ref, z_out_ref = refs[np_:]
    x = _combine(p_refs, takes, cnt_ref, z_ref)
    y_ref[...] = jnp.dot(x, wl_ref[...], preferred_element_type=jnp.float32)
    z_out_ref[...] = (jnp.dot(x, wr_ref[...],
                              preferred_element_type=jnp.float32) + bl_ref[...])


def _tc_mid(p_parts, takes, cnt, z, wl_pad, wr, bl):
    d_prev = z.shape[1]
    wp, d_out = wl_pad.shape[1], wr.shape[1]
    p_specs = [pl.BlockSpec((NC, _BN, p.shape[2]), lambda i: (0, i, 0))
               for p in p_parts]
    return pl.pallas_call(
        functools.partial(_tc_mid_body, tuple(takes)),
        grid=(_GRID,),
        in_specs=p_specs + [
            pl.BlockSpec((_BN, NW), lambda i: (i, 0)),
            pl.BlockSpec((_BN, d_prev), lambda i: (i, 0)),
            pl.BlockSpec((d_prev, wp), lambda i: (0, 0)),
            pl.BlockSpec((d_prev, d_out), lambda i: (0, 0)),
            pl.BlockSpec((1, d_out), lambda i: (0, 0)),
        ],
        out_specs=[
            pl.BlockSpec((_BN, wp), lambda i: (i, 0)),
            pl.BlockSpec((_BN, d_out), lambda i: (i, 0)),
        ],
        out_shape=[
            jax.ShapeDtypeStruct((N, wp), jnp.float32),
            jax.ShapeDtypeStruct((N, d_out), jnp.float32),
        ],
    )(*p_parts, cnt, z, wl_pad, wr, bl)


def _tc_last_body(p_ref, cnt_ref, z_ref, out_ref):
    cnt = jnp.sum(cnt_ref[...], axis=1, keepdims=True)
    inv = 1.0 / jnp.maximum(cnt, 1.0)
    agg = jnp.sum(p_ref[...], axis=1, keepdims=True)
    out_ref[...] = agg * inv + z_ref[...]


def _tc_last(p, cnt, z):
    return pl.pallas_call(
        _tc_last_body,
        grid=(_GRID,),
        in_specs=[
            pl.BlockSpec((_BN, NW), lambda i: (i, 0)),
            pl.BlockSpec((_BN, NW), lambda i: (i, 0)),
            pl.BlockSpec((_BN, D4), lambda i: (i, 0)),
        ],
        out_specs=pl.BlockSpec((_BN, D4), lambda i: (i, 0)),
        out_shape=jax.ShapeDtypeStruct((N, D4), jnp.float32),
    )(p, cnt, z)


def _pad_w(w, width):
    return jnp.pad(w, ((0, 0), (0, width - w.shape[1])))


def kernel(x, edge_index, Wl1, bl1, Wr1, Wl2, bl2, Wr2, Wl3, bl3, Wr3,
           Wl4, bl4, Wr4):
    src = edge_index[0].astype(jnp.int32)
    dst = edge_index[1].astype(jnp.int32)
    pad = E_PAD - E
    src = jnp.concatenate([src, jnp.zeros((pad,), jnp.int32)])
    dst = jnp.concatenate([dst, jnp.full((pad,), N, jnp.int32)])
    src = src.reshape(E_PAD // CHUNK, CHUNK)
    dst = dst.reshape(E_PAD // CHUNK, CHUNK)

    zeros = jnp.zeros((N_PAD, W2), jnp.float32)
    zrow = jnp.zeros((N_PAD,), jnp.float32)

    wla = Wl1[:, :W1A]
    wlb = _pad_w(Wl1[:, W1A:], W1B)

    y1a, y1b, z1 = _tc_first(x, wla, wlb, Wr1, bl1.reshape(1, -1))
    p1a, cnt = _sc_layer1a(y1a, src, dst, zeros[:, :W1A], zrow)
    p1b = _sc_layer1b(y1b, src, dst, zeros[:, :W1B])
    cnt = cnt.T[:N]

    y2, z2 = _tc_mid([p1a[:, :N], p1b[:, :N]], (W1A, D1 - W1A), cnt, z1,
                     _pad_w(Wl2, W2), Wr2, bl2.reshape(1, -1))
    p2 = _sc_layer2(y2, src, dst, zeros)

    y3, z3 = _tc_mid([p2[:, :N]], (D2,), cnt, z2,
                     _pad_w(Wl3, W3), Wr3, bl3.reshape(1, -1))
    p3 = _sc_layer3(y3, src, dst, zeros[:, :W3])

    y4, z4 = _tc_mid([p3[:, :N]], (D3,), cnt, z3,
                     Wl4, Wr4, bl4.reshape(1, -1))
    p4 = _sc_layer4(y4.reshape(N), src, dst, zrow)

    return _tc_last(p4.T[:N], cnt, z4)
